# baseline (speedup 1.0000x reference)
import functools

import jax
import jax.numpy as jnp
from jax import lax
from jax.experimental import pallas as pl
from jax.experimental.pallas import tpu as pltpu
from jax.experimental.pallas import tpu_sc as plsc

BATCH = 16384
EMB_DIM = 64
LANES = 16
NUM_ENT = 1000000
QROWS = 262144
ABLK = 4096

try:
    _info = plsc.get_sparse_core_info()
    NUM_CORES, NUM_SUBCORES = _info.num_cores, _info.num_subcores
except Exception:
    NUM_CORES, NUM_SUBCORES = 2, 16

NUM_WORKERS = NUM_CORES * NUM_SUBCORES
ROWS_PER_WORKER = BATCH // NUM_WORKERS
CHUNK = 128
NCHUNKS = ROWS_PER_WORKER // CHUNK
QBLKS = QROWS // ABLK
NINBLK = NUM_ENT // ABLK


QSCALE = float(2 ** 19)
QBIAS = 2048.0


def _pack_body(q0_ref, q1_ref, q2_ref, q3_ref, out_ref):
    r = lax.broadcasted_iota(jnp.int32, (EMB_DIM, EMB_DIM), 0)
    c = lax.broadcasted_iota(jnp.int32, (EMB_DIM, EMB_DIM), 1)
    iden = (r == c).astype(jnp.float32)
    dn = (((0,), (0,)), ((), ()))
    half = EMB_DIM // 2
    for q, ref in enumerate((q0_ref, q1_ref, q2_ref, q3_ref)):
        y = lax.dot_general(ref[...], iden, dn,
                            preferred_element_type=jnp.float32)
        qi = lax.convert_element_type(y * QSCALE + (QBIAS + 0.5), jnp.int32)
        w = qi[:, 0:half] + qi[:, half:EMB_DIM] * 4096
        out_ref[:, 32 * q:32 * (q + 1)] = lax.convert_element_type(
            w, jnp.float32)


def _pack(entT):
    def in_spec(q):
        return pl.BlockSpec(
            (EMB_DIM, ABLK),
            lambda k, _q=q: (0, jnp.minimum(_q * QBLKS + k, NINBLK)))

    return pl.pallas_call(
        _pack_body,
        grid=(QBLKS,),
        in_specs=[in_spec(0), in_spec(1), in_spec(2), in_spec(3)],
        out_specs=pl.BlockSpec((ABLK, 2 * EMB_DIM), lambda k: (k, 0)),
        out_shape=jax.ShapeDtypeStruct((QROWS, 2 * EMB_DIM), jnp.float32),
    )(entT, entT, entT, entT)


def _body(heads_hbm, rels_hbm, tails_hbm, ent_hbm, rel_hbm, out_hbm,
          idx_h, idx_r, idx_t, g_h, g_r, g_t,
          h0, h1, r0, r1, t0, t1, out_v, sems):
    wid = lax.axis_index("s") * NUM_CORES + lax.axis_index("c")
    base = wid * ROWS_PER_WORKER

    pltpu.sync_copy(heads_hbm.at[pl.ds(base, ROWS_PER_WORKER)], idx_h)
    pltpu.sync_copy(rels_hbm.at[pl.ds(base, ROWS_PER_WORKER)], idx_r)
    pltpu.sync_copy(tails_hbm.at[pl.ds(base, ROWS_PER_WORKER)], idx_t)

    for s in range(ROWS_PER_WORKER // LANES):
        sl = pl.ds(s * LANES, LANES)
        g_h[sl] = idx_h[sl] & (QROWS - 1)
        g_t[sl] = idx_t[sl] & (QROWS - 1)
        g_r[sl] = lax.shift_right_logical(idx_r[sl], 1)

    hbuf = (h0, h1)
    rbuf = (r0, r1)
    tbuf = (t0, t1)

    def fire(c):
        sl = pl.ds(c * CHUNK, CHUNK)
        s = c % 2
        return (
            pltpu.async_copy(ent_hbm.at[g_h.at[sl]], hbuf[s], sems[3 * s]),
            pltpu.async_copy(rel_hbm.at[g_r.at[sl]], rbuf[s], sems[3 * s + 1]),
            pltpu.async_copy(ent_hbm.at[g_t.at[sl]], tbuf[s], sems[3 * s + 2]),
        )

    lane = lax.iota(jnp.int32, LANES)
    bfly = [(lane ^ m).reshape(LANES, 1) for m in (1, 2, 4, 8)]
    _gdn = lax.GatherDimensionNumbers(
        offset_dims=(), collapsed_slice_dims=(0,), start_index_map=(0,))

    def _shuffle(x, idx):
        return lax.gather(x, idx, _gdn, (1,),
                          mode=lax.GatherScatterMode.PROMISE_IN_BOUNDS)

    inv4096 = 1.0 / 4096.0
    descale = 1.0 / QSCALE

    def _halves(w):
        hv = lax.convert_element_type(
            lax.convert_element_type(w * inv4096, jnp.int32), jnp.float32)
        av = w - hv * 4096.0
        return (av - QBIAS) * descale, (hv - QBIAS) * descale

    def compute(c):
        s = c % 2
        hb, rb, tb = hbuf[s], rbuf[s], tbuf[s]

        def group_step(g, carry):
            row0 = c * CHUNK + g * LANES
            gsl = pl.ds(row0, LANES)
            ivh = (idx_h[gsl] >> 18) << 5
            ivr = (idx_r[gsl] & 1) * EMB_DIM
            ivt = (idx_t[gsl] >> 18) << 5
            vec = jnp.zeros((LANES,), jnp.float32)
            for j in range(LANES):
                jj = g * LANES + j
                off_h = ivh[j]
                off_r = ivr[j]
                off_t = ivt[j]
                hw0 = hb[jj, pl.ds(off_h, LANES)]
                hw1 = hb[jj, pl.ds(off_h + LANES, LANES)]
                tw0 = tb[jj, pl.ds(off_t, LANES)]
                tw1 = tb[jj, pl.ds(off_t + LANES, LANES)]
                h0c, h2c = _halves(hw0)
                h1c, h3c = _halves(hw1)
                t0c, t2c = _halves(tw0)
                t1c, t3c = _halves(tw1)
                acc = None
                for k, (hc, tc) in enumerate(
                        ((h0c, t0c), (h1c, t1c), (h2c, t2c), (h3c, t3c))):
                    d = hc + rb[jj, pl.ds(off_r + k * LANES, LANES)] - tc
                    sq = d * d
                    acc = sq if acc is None else acc + sq
                for m in bfly:
                    acc = acc + _shuffle(acc, m)
                vec = jnp.where(lane == j, acc, vec)
            out_v[pl.ds(row0, LANES)] = vec
            return carry

        lax.fori_loop(0, CHUNK // LANES, group_step, 0)

    inflight = {0: fire(0), 1: fire(1)}
    for c in range(NCHUNKS):
        for cp in inflight.pop(c):
            cp.wait()
        compute(c)
        if c + 2 < NCHUNKS:
            inflight[c + 2] = fire(c + 2)

    pltpu.sync_copy(out_v, out_hbm.at[pl.ds(base, ROWS_PER_WORKER)])


@functools.partial(jax.jit, static_argnums=())
def _transe_sc(heads, rels, tails, entT, rel2):
    entP = _pack(entT)
    mesh = plsc.VectorSubcoreMesh(core_axis_name="c", subcore_axis_name="s")
    return pl.kernel(
        _body,
        mesh=mesh,
        compiler_params=pltpu.CompilerParams(use_tc_tiling_on_sc=True),
        out_type=jax.ShapeDtypeStruct((BATCH,), jnp.float32),
        scratch_types=[
            pltpu.VMEM((ROWS_PER_WORKER,), jnp.int32),
            pltpu.VMEM((ROWS_PER_WORKER,), jnp.int32),
            pltpu.VMEM((ROWS_PER_WORKER,), jnp.int32),
            pltpu.VMEM((ROWS_PER_WORKER,), jnp.int32),
            pltpu.VMEM((ROWS_PER_WORKER,), jnp.int32),
            pltpu.VMEM((ROWS_PER_WORKER,), jnp.int32),
            pltpu.VMEM((CHUNK, 2 * EMB_DIM), jnp.float32),
            pltpu.VMEM((CHUNK, 2 * EMB_DIM), jnp.float32),
            pltpu.VMEM((CHUNK, 2 * EMB_DIM), jnp.float32),
            pltpu.VMEM((CHUNK, 2 * EMB_DIM), jnp.float32),
            pltpu.VMEM((CHUNK, 2 * EMB_DIM), jnp.float32),
            pltpu.VMEM((CHUNK, 2 * EMB_DIM), jnp.float32),
            pltpu.VMEM((ROWS_PER_WORKER,), jnp.float32),
            [pltpu.SemaphoreType.DMA] * 6,
        ],
    )(heads, rels, tails, entP, rel2)


def kernel(heads, relations, tails, ent_embeddings, rel_embeddings):
    rel2 = rel_embeddings.reshape(rel_embeddings.shape[0] // 2, 2 * EMB_DIM)
    return _transe_sc(heads.astype(jnp.int32), relations.astype(jnp.int32),
                      tails.astype(jnp.int32), ent_embeddings.T, rel2)


# speedup vs baseline: 1.1796x; 1.1796x over previous
import functools

import jax
import jax.numpy as jnp
from jax import lax
from jax.experimental import pallas as pl
from jax.experimental.pallas import tpu as pltpu
from jax.experimental.pallas import tpu_sc as plsc

BATCH = 16384
EMB_DIM = 64
LANES = 16
NUM_ENT = 1000000
QROWS = 262144
ABLK = 4096

try:
    _info = plsc.get_sparse_core_info()
    NUM_CORES, NUM_SUBCORES = _info.num_cores, _info.num_subcores
except Exception:
    NUM_CORES, NUM_SUBCORES = 2, 16

NUM_WORKERS = NUM_CORES * NUM_SUBCORES
ROWS_PER_WORKER = BATCH // NUM_WORKERS
CHUNK = 128
NCHUNKS = ROWS_PER_WORKER // CHUNK
QBLKS = QROWS // ABLK
NINBLK = NUM_ENT // ABLK


QSCALE = float(2 ** 19)
QBIAS = 2048.0


def _pack_body(q0_ref, q1_ref, q2_ref, q3_ref, out_ref):
    r = lax.broadcasted_iota(jnp.int32, (EMB_DIM, EMB_DIM), 0)
    c = lax.broadcasted_iota(jnp.int32, (EMB_DIM, EMB_DIM), 1)
    iden = jnp.where(r == c, QSCALE, 0.0).astype(jnp.bfloat16)
    dn = (((0,), (0,)), ((), ()))
    half = EMB_DIM // 2
    for q, ref in enumerate((q0_ref, q1_ref, q2_ref, q3_ref)):
        y = lax.dot_general(ref[...].astype(jnp.bfloat16), iden, dn,
                            preferred_element_type=jnp.float32)
        qi = lax.convert_element_type(y + (QBIAS + 0.5), jnp.int32)
        w = qi[:, 0:half] + (qi[:, half:EMB_DIM] << 12)
        out_ref[:, 32 * q:32 * (q + 1)] = lax.convert_element_type(
            w, jnp.float32)


def _pack(entT):
    def in_spec(q):
        return pl.BlockSpec(
            (EMB_DIM, ABLK),
            lambda k, _q=q: (0, jnp.minimum(_q * QBLKS + k, NINBLK)))

    return pl.pallas_call(
        _pack_body,
        grid=(QBLKS,),
        in_specs=[in_spec(0), in_spec(1), in_spec(2), in_spec(3)],
        out_specs=pl.BlockSpec((ABLK, 2 * EMB_DIM), lambda k: (k, 0)),
        out_shape=jax.ShapeDtypeStruct((QROWS, 2 * EMB_DIM), jnp.float32),
    )(entT, entT, entT, entT)


def _body(heads_hbm, rels_hbm, tails_hbm, ent_hbm, rel_hbm, out_hbm,
          idx_h, idx_r, idx_t, g_h, g_r, g_t,
          h0, h1, r0, r1, t0, t1, out_v, sems):
    wid = lax.axis_index("s") * NUM_CORES + lax.axis_index("c")
    base = wid * ROWS_PER_WORKER

    pltpu.sync_copy(heads_hbm.at[pl.ds(base, ROWS_PER_WORKER)], idx_h)
    pltpu.sync_copy(rels_hbm.at[pl.ds(base, ROWS_PER_WORKER)], idx_r)
    pltpu.sync_copy(tails_hbm.at[pl.ds(base, ROWS_PER_WORKER)], idx_t)

    for s in range(ROWS_PER_WORKER // LANES):
        sl = pl.ds(s * LANES, LANES)
        g_h[sl] = idx_h[sl] & (QROWS - 1)
        g_t[sl] = idx_t[sl] & (QROWS - 1)
        g_r[sl] = lax.shift_right_logical(idx_r[sl], 1)

    hbuf = (h0, h1)
    rbuf = (r0, r1)
    tbuf = (t0, t1)

    def fire(c):
        sl = pl.ds(c * CHUNK, CHUNK)
        s = c % 2
        return (
            pltpu.async_copy(ent_hbm.at[g_h.at[sl]], hbuf[s], sems[3 * s]),
            pltpu.async_copy(rel_hbm.at[g_r.at[sl]], rbuf[s], sems[3 * s + 1]),
            pltpu.async_copy(ent_hbm.at[g_t.at[sl]], tbuf[s], sems[3 * s + 2]),
        )

    lane = lax.iota(jnp.int32, LANES)
    bfly = [(lane ^ m).reshape(LANES, 1) for m in (1, 2, 4, 8)]
    _gdn = lax.GatherDimensionNumbers(
        offset_dims=(), collapsed_slice_dims=(0,), start_index_map=(0,))

    def _shuffle(x, idx):
        return lax.gather(x, idx, _gdn, (1,),
                          mode=lax.GatherScatterMode.PROMISE_IN_BOUNDS)

    inv4096 = 1.0 / 4096.0
    descale = 1.0 / QSCALE

    def _halves(w):
        hv = lax.convert_element_type(
            lax.convert_element_type(w * inv4096, jnp.int32), jnp.float32)
        av = w - hv * 4096.0
        return (av - QBIAS) * descale, (hv - QBIAS) * descale

    def compute(c):
        s = c % 2
        hb, rb, tb = hbuf[s], rbuf[s], tbuf[s]

        def group_step(g, carry):
            row0 = c * CHUNK + g * LANES
            gsl = pl.ds(row0, LANES)
            ivh = (idx_h[gsl] >> 18) << 5
            ivr = (idx_r[gsl] & 1) * EMB_DIM
            ivt = (idx_t[gsl] >> 18) << 5
            vec = jnp.zeros((LANES,), jnp.float32)
            for j in range(LANES):
                jj = g * LANES + j
                off_h = ivh[j]
                off_r = ivr[j]
                off_t = ivt[j]
                hw0 = hb[jj, pl.ds(off_h, LANES)]
                hw1 = hb[jj, pl.ds(off_h + LANES, LANES)]
                tw0 = tb[jj, pl.ds(off_t, LANES)]
                tw1 = tb[jj, pl.ds(off_t + LANES, LANES)]
                h0c, h2c = _halves(hw0)
                h1c, h3c = _halves(hw1)
                t0c, t2c = _halves(tw0)
                t1c, t3c = _halves(tw1)
                acc = None
                for k, (hc, tc) in enumerate(
                        ((h0c, t0c), (h1c, t1c), (h2c, t2c), (h3c, t3c))):
                    d = hc + rb[jj, pl.ds(off_r + k * LANES, LANES)] - tc
                    sq = d * d
                    acc = sq if acc is None else acc + sq
                for m in bfly:
                    acc = acc + _shuffle(acc, m)
                vec = jnp.where(lane == j, acc, vec)
            out_v[pl.ds(row0, LANES)] = vec
            return carry

        lax.fori_loop(0, CHUNK // LANES, group_step, 0)

    inflight = {0: fire(0), 1: fire(1)}
    for c in range(NCHUNKS):
        for cp in inflight.pop(c):
            cp.wait()
        compute(c)
        if c + 2 < NCHUNKS:
            inflight[c + 2] = fire(c + 2)

    pltpu.sync_copy(out_v, out_hbm.at[pl.ds(base, ROWS_PER_WORKER)])


@functools.partial(jax.jit, static_argnums=())
def _transe_sc(heads, rels, tails, entT, rel2):
    entP = _pack(entT)
    mesh = plsc.VectorSubcoreMesh(core_axis_name="c", subcore_axis_name="s")
    return pl.kernel(
        _body,
        mesh=mesh,
        compiler_params=pltpu.CompilerParams(use_tc_tiling_on_sc=True),
        out_type=jax.ShapeDtypeStruct((BATCH,), jnp.float32),
        scratch_types=[
            pltpu.VMEM((ROWS_PER_WORKER,), jnp.int32),
            pltpu.VMEM((ROWS_PER_WORKER,), jnp.int32),
            pltpu.VMEM((ROWS_PER_WORKER,), jnp.int32),
            pltpu.VMEM((ROWS_PER_WORKER,), jnp.int32),
            pltpu.VMEM((ROWS_PER_WORKER,), jnp.int32),
            pltpu.VMEM((ROWS_PER_WORKER,), jnp.int32),
            pltpu.VMEM((CHUNK, 2 * EMB_DIM), jnp.float32),
            pltpu.VMEM((CHUNK, 2 * EMB_DIM), jnp.float32),
            pltpu.VMEM((CHUNK, 2 * EMB_DIM), jnp.float32),
            pltpu.VMEM((CHUNK, 2 * EMB_DIM), jnp.float32),
            pltpu.VMEM((CHUNK, 2 * EMB_DIM), jnp.float32),
            pltpu.VMEM((CHUNK, 2 * EMB_DIM), jnp.float32),
            pltpu.VMEM((ROWS_PER_WORKER,), jnp.float32),
            [pltpu.SemaphoreType.DMA] * 6,
        ],
    )(heads, rels, tails, entP, rel2)


def kernel(heads, relations, tails, ent_embeddings, rel_embeddings):
    rel2 = rel_embeddings.reshape(rel_embeddings.shape[0] // 2, 2 * EMB_DIM)
    return _transe_sc(heads.astype(jnp.int32), relations.astype(jnp.int32),
                      tails.astype(jnp.int32), ent_embeddings.T, rel2)


# speedup vs baseline: 1.6203x; 1.3736x over previous
import functools

import jax
import jax.numpy as jnp
from jax import lax
from jax.experimental import pallas as pl
from jax.experimental.pallas import tpu as pltpu
from jax.experimental.pallas import tpu_sc as plsc

BATCH = 16384
EMB_DIM = 64
LANES = 16
NUM_ENT = 1000000
SPLIT = 524288
PACKED_ROWS = SPLIT
ABLK = 8192

try:
    _info = plsc.get_sparse_core_info()
    NUM_CORES, NUM_SUBCORES = _info.num_cores, _info.num_subcores
except Exception:
    NUM_CORES, NUM_SUBCORES = 2, 16

NUM_WORKERS = NUM_CORES * NUM_SUBCORES
ROWS_PER_WORKER = BATCH // NUM_WORKERS
CHUNK = 128
NCHUNKS = ROWS_PER_WORKER // CHUNK
NLBLK = NUM_ENT // ABLK


def _pack_body(left_ref, right_ref, out_ref):
    r = lax.broadcasted_iota(jnp.int32, (EMB_DIM, EMB_DIM), 0)
    c = lax.broadcasted_iota(jnp.int32, (EMB_DIM, EMB_DIM), 1)
    iden = (r == c).astype(jnp.float32)
    dn = (((0,), (0,)), ((), ()))
    out_ref[:, 0:EMB_DIM] = lax.dot_general(
        left_ref[...], iden, dn, preferred_element_type=jnp.float32)
    out_ref[:, EMB_DIM:2 * EMB_DIM] = lax.dot_general(
        right_ref[...], iden, dn, preferred_element_type=jnp.float32)


def _pack(entT):
    grid = (SPLIT // ABLK,)
    return pl.pallas_call(
        _pack_body,
        grid=grid,
        in_specs=[
            pl.BlockSpec((EMB_DIM, ABLK), lambda k: (0, k)),
            pl.BlockSpec((EMB_DIM, ABLK),
                         lambda k: (0, jnp.minimum(k + SPLIT // ABLK,
                                                   NUM_ENT // ABLK))),
        ],
        out_specs=pl.BlockSpec((ABLK, 2 * EMB_DIM), lambda k: (k, 0)),
        out_shape=jax.ShapeDtypeStruct((PACKED_ROWS, 2 * EMB_DIM),
                                       jnp.float32),
    )(entT, entT)


def _body(heads_hbm, rels_hbm, tails_hbm, ent_hbm, rel_hbm, out_hbm,
          idx_h, idx_r, idx_t, g_h, g_r, g_t,
          h0, h1, r0, r1, t0, t1, out_v, sems):
    wid = lax.axis_index("s") * NUM_CORES + lax.axis_index("c")
    base = wid * ROWS_PER_WORKER

    pltpu.sync_copy(heads_hbm.at[pl.ds(base, ROWS_PER_WORKER)], idx_h)
    pltpu.sync_copy(rels_hbm.at[pl.ds(base, ROWS_PER_WORKER)], idx_r)
    pltpu.sync_copy(tails_hbm.at[pl.ds(base, ROWS_PER_WORKER)], idx_t)

    for s in range(ROWS_PER_WORKER // LANES):
        sl = pl.ds(s * LANES, LANES)
        ih = idx_h[sl]
        it = idx_t[sl]
        g_h[sl] = jnp.where(ih >= SPLIT, ih - SPLIT, ih)
        g_t[sl] = jnp.where(it >= SPLIT, it - SPLIT, it)
        g_r[sl] = lax.shift_right_logical(idx_r[sl], 1)

    hbuf = (h0, h1)
    rbuf = (r0, r1)
    tbuf = (t0, t1)

    def fire(c):
        sl = pl.ds(c * CHUNK, CHUNK)
        s = c % 2
        return (
            pltpu.async_copy(ent_hbm.at[g_h.at[sl]], hbuf[s], sems[3 * s]),
            pltpu.async_copy(rel_hbm.at[g_r.at[sl]], rbuf[s], sems[3 * s + 1]),
            pltpu.async_copy(ent_hbm.at[g_t.at[sl]], tbuf[s], sems[3 * s + 2]),
        )

    lane = lax.iota(jnp.int32, LANES)
    bfly = [(lane ^ m).reshape(LANES, 1) for m in (1, 2, 4, 8)]
    _gdn = lax.GatherDimensionNumbers(
        offset_dims=(), collapsed_slice_dims=(0,), start_index_map=(0,))

    def _shuffle(x, idx):
        return lax.gather(x, idx, _gdn, (1,),
                          mode=lax.GatherScatterMode.PROMISE_IN_BOUNDS)

    def compute(c):
        s = c % 2
        hb, rb, tb = hbuf[s], rbuf[s], tbuf[s]

        def group_step(g, carry):
            row0 = c * CHUNK + g * LANES
            gsl = pl.ds(row0, LANES)
            ivh = jnp.where(idx_h[gsl] >= SPLIT, EMB_DIM, 0)
            ivr = (idx_r[gsl] & 1) * EMB_DIM
            ivt = jnp.where(idx_t[gsl] >= SPLIT, EMB_DIM, 0)
            vec = jnp.zeros((LANES,), jnp.float32)
            for j in range(LANES):
                jj = g * LANES + j
                off_h = ivh[j]
                off_r = ivr[j]
                off_t = ivt[j]
                acc = None
                for k in range(EMB_DIM // LANES):
                    d = (hb[jj, pl.ds(off_h + k * LANES, LANES)]
                         + rb[jj, pl.ds(off_r + k * LANES, LANES)]
                         - tb[jj, pl.ds(off_t + k * LANES, LANES)])
                    sq = d * d
                    acc = sq if acc is None else acc + sq
                for m in bfly:
                    acc = acc + _shuffle(acc, m)
                vec = jnp.where(lane == j, acc, vec)
            out_v[pl.ds(row0, LANES)] = vec
            return carry

        lax.fori_loop(0, CHUNK // LANES, group_step, 0)

    inflight = {0: fire(0), 1: fire(1)}
    for c in range(NCHUNKS):
        for cp in inflight.pop(c):
            cp.wait()
        compute(c)
        if c + 2 < NCHUNKS:
            inflight[c + 2] = fire(c + 2)

    pltpu.sync_copy(out_v, out_hbm.at[pl.ds(base, ROWS_PER_WORKER)])


@functools.partial(jax.jit, static_argnums=())
def _transe_sc(heads, rels, tails, entT, rel2):
    ent2 = _pack(entT)
    mesh = plsc.VectorSubcoreMesh(core_axis_name="c", subcore_axis_name="s")
    return pl.kernel(
        _body,
        mesh=mesh,
        compiler_params=pltpu.CompilerParams(use_tc_tiling_on_sc=True),
        out_type=jax.ShapeDtypeStruct((BATCH,), jnp.float32),
        scratch_types=[
            pltpu.VMEM((ROWS_PER_WORKER,), jnp.int32),
            pltpu.VMEM((ROWS_PER_WORKER,), jnp.int32),
            pltpu.VMEM((ROWS_PER_WORKER,), jnp.int32),
            pltpu.VMEM((ROWS_PER_WORKER,), jnp.int32),
            pltpu.VMEM((ROWS_PER_WORKER,), jnp.int32),
            pltpu.VMEM((ROWS_PER_WORKER,), jnp.int32),
            pltpu.VMEM((CHUNK, 2 * EMB_DIM), jnp.float32),
            pltpu.VMEM((CHUNK, 2 * EMB_DIM), jnp.float32),
            pltpu.VMEM((CHUNK, 2 * EMB_DIM), jnp.float32),
            pltpu.VMEM((CHUNK, 2 * EMB_DIM), jnp.float32),
            pltpu.VMEM((CHUNK, 2 * EMB_DIM), jnp.float32),
            pltpu.VMEM((CHUNK, 2 * EMB_DIM), jnp.float32),
            pltpu.VMEM((ROWS_PER_WORKER,), jnp.float32),
            [pltpu.SemaphoreType.DMA] * 6,
        ],
    )(heads, rels, tails, ent2, rel2)


def kernel(heads, relations, tails, ent_embeddings, rel_embeddings):
    rel2 = rel_embeddings.reshape(rel_embeddings.shape[0] // 2, 2 * EMB_DIM)
    return _transe_sc(heads.astype(jnp.int32), relations.astype(jnp.int32),
                      tails.astype(jnp.int32), ent_embeddings.T, rel2)


# speedup vs baseline: 1.7094x; 1.0550x over previous
import functools

import jax
import jax.numpy as jnp
from jax import lax
from jax.experimental import pallas as pl
from jax.experimental.pallas import tpu as pltpu
from jax.experimental.pallas import tpu_sc as plsc

BATCH = 16384
EMB_DIM = 64
LANES = 16
NUM_ENT = 1000000
SPLIT = 524288
PACKED_ROWS = SPLIT
ABLK = 16384

try:
    _info = plsc.get_sparse_core_info()
    NUM_CORES, NUM_SUBCORES = _info.num_cores, _info.num_subcores
except Exception:
    NUM_CORES, NUM_SUBCORES = 2, 16

NUM_WORKERS = NUM_CORES * NUM_SUBCORES
ROWS_PER_WORKER = BATCH // NUM_WORKERS
CHUNK = 128
NCHUNKS = ROWS_PER_WORKER // CHUNK
NLBLK = NUM_ENT // ABLK


def _pack_body(left_ref, right_ref, out_ref):
    r = lax.broadcasted_iota(jnp.int32, (EMB_DIM, EMB_DIM), 0)
    c = lax.broadcasted_iota(jnp.int32, (EMB_DIM, EMB_DIM), 1)
    iden = (r == c).astype(jnp.float32)
    dn = (((0,), (0,)), ((), ()))
    out_ref[:, 0:EMB_DIM] = lax.dot_general(
        left_ref[...], iden, dn, preferred_element_type=jnp.float32)
    out_ref[:, EMB_DIM:2 * EMB_DIM] = lax.dot_general(
        right_ref[...], iden, dn, preferred_element_type=jnp.float32)


def _pack(entT):
    grid = (SPLIT // ABLK,)
    return pl.pallas_call(
        _pack_body,
        grid=grid,
        in_specs=[
            pl.BlockSpec((EMB_DIM, ABLK), lambda k: (0, k)),
            pl.BlockSpec((EMB_DIM, ABLK),
                         lambda k: (0, jnp.minimum(k + SPLIT // ABLK,
                                                   NUM_ENT // ABLK))),
        ],
        out_specs=pl.BlockSpec((ABLK, 2 * EMB_DIM), lambda k: (k, 0)),
        out_shape=jax.ShapeDtypeStruct((PACKED_ROWS, 2 * EMB_DIM),
                                       jnp.float32),
    )(entT, entT)


def _body(heads_hbm, rels_hbm, tails_hbm, ent_hbm, rel_hbm, out_hbm,
          idx_h, idx_r, idx_t, g_h, g_r, g_t,
          h0, h1, r0, r1, t0, t1, out_v, sems):
    wid = lax.axis_index("s") * NUM_CORES + lax.axis_index("c")
    base = wid * ROWS_PER_WORKER

    pltpu.sync_copy(heads_hbm.at[pl.ds(base, ROWS_PER_WORKER)], idx_h)
    pltpu.sync_copy(rels_hbm.at[pl.ds(base, ROWS_PER_WORKER)], idx_r)
    pltpu.sync_copy(tails_hbm.at[pl.ds(base, ROWS_PER_WORKER)], idx_t)

    for s in range(ROWS_PER_WORKER // LANES):
        sl = pl.ds(s * LANES, LANES)
        ih = idx_h[sl]
        it = idx_t[sl]
        g_h[sl] = jnp.where(ih >= SPLIT, ih - SPLIT, ih)
        g_t[sl] = jnp.where(it >= SPLIT, it - SPLIT, it)
        g_r[sl] = lax.shift_right_logical(idx_r[sl], 1)

    hbuf = (h0, h1)
    rbuf = (r0, r1)
    tbuf = (t0, t1)

    def fire(c):
        sl = pl.ds(c * CHUNK, CHUNK)
        s = c % 2
        return (
            pltpu.async_copy(ent_hbm.at[g_h.at[sl]], hbuf[s], sems[3 * s]),
            pltpu.async_copy(rel_hbm.at[g_r.at[sl]], rbuf[s], sems[3 * s + 1]),
            pltpu.async_copy(ent_hbm.at[g_t.at[sl]], tbuf[s], sems[3 * s + 2]),
        )

    lane = lax.iota(jnp.int32, LANES)
    bfly = [(lane ^ m).reshape(LANES, 1) for m in (1, 2, 4, 8)]
    _gdn = lax.GatherDimensionNumbers(
        offset_dims=(), collapsed_slice_dims=(0,), start_index_map=(0,))

    def _shuffle(x, idx):
        return lax.gather(x, idx, _gdn, (1,),
                          mode=lax.GatherScatterMode.PROMISE_IN_BOUNDS)

    def compute(c):
        s = c % 2
        hb, rb, tb = hbuf[s], rbuf[s], tbuf[s]

        def group_step(g, carry):
            row0 = c * CHUNK + g * LANES
            gsl = pl.ds(row0, LANES)
            ivh = jnp.where(idx_h[gsl] >= SPLIT, EMB_DIM, 0)
            ivr = (idx_r[gsl] & 1) * EMB_DIM
            ivt = jnp.where(idx_t[gsl] >= SPLIT, EMB_DIM, 0)
            vec = jnp.zeros((LANES,), jnp.float32)
            for j in range(LANES):
                jj = g * LANES + j
                off_h = ivh[j]
                off_r = ivr[j]
                off_t = ivt[j]
                acc = None
                for k in range(EMB_DIM // LANES):
                    d = (hb[jj, pl.ds(off_h + k * LANES, LANES)]
                         + rb[jj, pl.ds(off_r + k * LANES, LANES)]
                         - tb[jj, pl.ds(off_t + k * LANES, LANES)])
                    sq = d * d
                    acc = sq if acc is None else acc + sq
                for m in bfly:
                    acc = acc + _shuffle(acc, m)
                vec = jnp.where(lane == j, acc, vec)
            out_v[pl.ds(row0, LANES)] = vec
            return carry

        lax.fori_loop(0, CHUNK // LANES, group_step, 0)

    inflight = {0: fire(0), 1: fire(1)}
    for c in range(NCHUNKS):
        for cp in inflight.pop(c):
            cp.wait()
        compute(c)
        if c + 2 < NCHUNKS:
            inflight[c + 2] = fire(c + 2)

    pltpu.sync_copy(out_v, out_hbm.at[pl.ds(base, ROWS_PER_WORKER)])


@functools.partial(jax.jit, static_argnums=())
def _transe_sc(heads, rels, tails, entT, rel2):
    ent2 = _pack(entT)
    mesh = plsc.VectorSubcoreMesh(core_axis_name="c", subcore_axis_name="s")
    return pl.kernel(
        _body,
        mesh=mesh,
        compiler_params=pltpu.CompilerParams(use_tc_tiling_on_sc=True),
        out_type=jax.ShapeDtypeStruct((BATCH,), jnp.float32),
        scratch_types=[
            pltpu.VMEM((ROWS_PER_WORKER,), jnp.int32),
            pltpu.VMEM((ROWS_PER_WORKER,), jnp.int32),
            pltpu.VMEM((ROWS_PER_WORKER,), jnp.int32),
            pltpu.VMEM((ROWS_PER_WORKER,), jnp.int32),
            pltpu.VMEM((ROWS_PER_WORKER,), jnp.int32),
            pltpu.VMEM((ROWS_PER_WORKER,), jnp.int32),
            pltpu.VMEM((CHUNK, 2 * EMB_DIM), jnp.float32),
            pltpu.VMEM((CHUNK, 2 * EMB_DIM), jnp.float32),
            pltpu.VMEM((CHUNK, 2 * EMB_DIM), jnp.float32),
            pltpu.VMEM((CHUNK, 2 * EMB_DIM), jnp.float32),
            pltpu.VMEM((CHUNK, 2 * EMB_DIM), jnp.float32),
            pltpu.VMEM((CHUNK, 2 * EMB_DIM), jnp.float32),
            pltpu.VMEM((ROWS_PER_WORKER,), jnp.float32),
            [pltpu.SemaphoreType.DMA] * 6,
        ],
    )(heads, rels, tails, ent2, rel2)


def kernel(heads, relations, tails, ent_embeddings, rel_embeddings):
    rel2 = rel_embeddings.reshape(rel_embeddings.shape[0] // 2, 2 * EMB_DIM)
    return _transe_sc(heads.astype(jnp.int32), relations.astype(jnp.int32),
                      tails.astype(jnp.int32), ent_embeddings.T, rel2)
